# baseline (speedup 1.0000x reference)
import functools

import jax
import jax.numpy as jnp
import numpy as np
from jax import lax
from jax.experimental import pallas as pl
from jax.experimental.pallas import tpu as pltpu
from jax.experimental.pallas import tpu_sc as plsc

NCORES = 2
NSUB = 16
NW = NCORES * NSUB
L = 16

_f32 = jnp.float32
_i32 = jnp.int32


def _pick_chunk(n, hi, lo=8):
    for k in range(hi, lo - 1, -8):
        if n % k == 0:
            return k
    raise ValueError(f"no chunk for {n}")


def _pick_div(n, hi):
    for k in range(hi, 0, -1):
        if n % k == 0:
            return k
    return 1


def _sc_mesh():
    return plsc.VectorSubcoreMesh(
        core_axis_name="c", subcore_axis_name="s",
        num_cores=NCORES, num_subcores=NSUB)


def _make_posgather(E, NSs, NAa):
    per_w = E // NW
    K = _pick_chunk(per_w, 2000)
    n_chunks = per_w // K
    n_grp = K // L

    @functools.partial(
        pl.kernel,
        out_type=[jax.ShapeDtypeStruct((E * 8,), _f32),
                  jax.ShapeDtypeStruct((E * 8,), _f32)],
        mesh=_sc_mesh(),
        compiler_params=pltpu.CompilerParams(needs_layout_passes=False),
        scratch_types=[
            pltpu.VMEM((2 * NSs,), _f32),
            pltpu.VMEM((2 * NAa,), _f32),
            [pltpu.VMEM((K,), _i32)] * 2,
            [pltpu.VMEM((K,), _i32)] * 2,
            [pltpu.VMEM((K,), _f32)] * 2,
            [pltpu.VMEM((K * 8,), _f32)] * 2,
            [pltpu.SemaphoreType.DMA] * 2,
            [pltpu.SemaphoreType.DMA] * 2,
        ],
    )
    def kern(ps_hbm, pa_hbm, a_src_hbm, a_dst_hbm, a_dis_hbm,
             s_src_hbm, s_dst_hbm, s_dis_hbm,
             out_a_hbm, out_s_hbm,
             ps_v, pa_v, srci_v, dsti_v, dis_v, stage_v, sem_in, sem_out):
        cid = lax.axis_index("c")
        sid = lax.axis_index("s")
        wid = sid * NCORES + cid
        pltpu.sync_copy(ps_hbm, ps_v)
        pltpu.sync_copy(pa_hbm, pa_v)
        lanes = lax.iota(_i32, L)
        zeros_f = jnp.zeros((L,), _f32)
        ones_i = jnp.ones((L,), _i32)

        def do_edges(src_hbm, dst_hbm, dis_hbm, out_hbm, src_tab):
            def issue_in(i, b):
                base = wid * per_w + i * K
                return (
                    pltpu.async_copy(src_hbm.at[pl.ds(base, K)],
                                     srci_v[b], sem_in[b]),
                    pltpu.async_copy(dst_hbm.at[pl.ds(base, K)],
                                     dsti_v[b], sem_in[b]),
                    pltpu.async_copy(dis_hbm.at[pl.ds(base, K)],
                                     dis_v[b], sem_in[b]),
                )

            pending_in = {0: issue_in(0, 0)}
            pending_out = {}
            for i in range(n_chunks):
                b = i % 2
                if i + 1 < n_chunks:
                    pending_in[i + 1] = issue_in(i + 1, 1 - b)
                for d in pending_in.pop(i):
                    d.wait()
                if i - 2 in pending_out:
                    pending_out.pop(i - 2).wait()

                def grp(g, carry):
                    rid8 = (lanes + g * L) * 8
                    srci = srci_v[b][pl.ds(g * L, L)] * 2
                    dsti = dsti_v[b][pl.ds(g * L, L)] * 2
                    sx = plsc.load_gather(src_tab, [srci])
                    sy = plsc.load_gather(src_tab, [srci + ones_i])
                    dx = plsc.load_gather(ps_v, [dsti])
                    dy = plsc.load_gather(ps_v, [dsti + ones_i])
                    dv = dis_v[b][pl.ds(g * L, L)]
                    for col, val in ((0, sx), (1, sy), (2, dx), (3, dy),
                                     (4, dv), (5, zeros_f), (6, zeros_f),
                                     (7, zeros_f)):
                        plsc.store_scatter(
                            stage_v[b], [rid8 + col], val)
                    return carry
                lax.fori_loop(0, n_grp, grp, 0)

                base = (wid * per_w + i * K) * 8
                pending_out[i] = pltpu.async_copy(
                    stage_v[b], out_hbm.at[pl.ds(base, K * 8)], sem_out[b])
            for d in pending_out.values():
                d.wait()

        do_edges(a_src_hbm, a_dst_hbm, a_dis_hbm, out_a_hbm, pa_v)
        do_edges(s_src_hbm, s_dst_hbm, s_dis_hbm, out_s_hbm, ps_v)

    return kern


def _make_scatter(E, NSs, with_count):
    K = 16
    per_w = E // NW
    assert per_w % K == 0
    n_chunks = per_w // K
    assert n_chunks >= 16
    NSP = -(-NSs // 128) * 128
    assert (NSP // NSUB) % 8 == 0
    rows_per_tile = NSP // NSUB
    ZR = 8

    NR = -(-(NSP // 128) // 8) * 8
    out_type = [jax.ShapeDtypeStruct((2 * NSP, 128), _f32)]
    scratch = [
        [pltpu.VMEM((K,), _i32)] * 4,
        [pltpu.VMEM((K,), _i32)] * 4,
        [pltpu.VMEM((K, 128), _f32)] * 2,
        [pltpu.VMEM((K, 128), _f32)] * 2,
        pltpu.VMEM_SHARED((NSP, 128), _f32),
        [pltpu.SemaphoreType.DMA] * 4,
        [pltpu.SemaphoreType.DMA] * 2,
        [pltpu.SemaphoreType.DMA] * 2,
        [pltpu.SemaphoreType.DMA] * 2,
    ]
    if with_count:
        out_type.append(jax.ShapeDtypeStruct((2 * NR, 128), _f32))
        scratch += [
            pltpu.VMEM((NR, 128), _f32),
            pltpu.VMEM((NR,), _i32),
            pltpu.VMEM_SHARED((NR, 128), _f32),
        ]

    @functools.partial(
        pl.kernel, out_type=out_type, mesh=_sc_mesh(),
        compiler_params=pltpu.CompilerParams(needs_layout_passes=False),
        scratch_types=scratch)
    def kern(feat_hbm, gate_hbm, src_hbm, dst_hbm, *rest):
        if with_count:
            (out_hbm, cnt_hbm, src_v, dst_v, rows_v, gate_v, acc,
             sem_idx, sem_rows, sem_gate, sem_sc,
             hist_v, iota_v, cacc) = rest
        else:
            (out_hbm, src_v, dst_v, rows_v, gate_v, acc,
             sem_idx, sem_rows, sem_gate, sem_sc) = rest
        cid = lax.axis_index("c")
        sid = lax.axis_index("s")
        wid = sid * NCORES + cid
        r0 = sid * rows_per_tile
        zf = jnp.zeros((L,), _f32)
        onef = jnp.ones((L,), _f32)
        lanes = lax.iota(_i32, L)

        def zero_rows0(j, carry):
            for c in range(8):
                rows_v[0][j, pl.ds(c * L, L)] = zf
            return carry
        lax.fori_loop(0, K, zero_rows0, 0)

        def zero_acc(j, carry):
            pltpu.sync_copy(rows_v[0].at[pl.ds(0, ZR)],
                            acc.at[pl.ds(r0 + j * ZR, ZR)])
            return carry
        lax.fori_loop(0, rows_per_tile // ZR, zero_acc, 0)

        if with_count:
            def zero_hist(j, carry):
                for c in range(8):
                    hist_v[j, pl.ds(c * L, L)] = zf
                return carry
            lax.fori_loop(0, NR, zero_hist, 0)
            for g in range(NR // L):
                iota_v[pl.ds(g * L, L)] = lanes + g * L

            @pl.when(sid < NR // ZR)
            def _():
                pltpu.sync_copy(rows_v[0].at[pl.ds(0, ZR)],
                                cacc.at[pl.ds(sid * ZR, ZR)])

        plsc.subcore_barrier()

        def issue_idx(i, q):
            base = wid * per_w + i * K
            pltpu.async_copy(src_hbm.at[pl.ds(base, K)], src_v[q],
                             sem_idx[q])
            pltpu.async_copy(dst_hbm.at[pl.ds(base, K)], dst_v[q],
                             sem_idx[q])

        def wait_idx(q):
            pltpu.make_async_copy(src_hbm.at[pl.ds(0, K)], src_v[q],
                                  sem_idx[q]).wait()
            pltpu.make_async_copy(dst_hbm.at[pl.ds(0, K)], dst_v[q],
                                  sem_idx[q]).wait()

        def issue_fetch(i, b, q):
            base = wid * per_w + i * K
            pltpu.async_copy(feat_hbm.at[src_v[q]], rows_v[b], sem_rows[b])
            pltpu.async_copy(gate_hbm.at[pl.ds(base, K)], gate_v[b],
                             sem_gate[b])

        def wait_fetch(b):
            pltpu.make_async_copy(feat_hbm.at[pl.ds(0, K)], rows_v[b],
                                  sem_rows[b]).wait()
            pltpu.make_async_copy(gate_hbm.at[pl.ds(0, K)], gate_v[b],
                                  sem_gate[b]).wait()

        def issue_scat(b, q):
            pltpu.async_copy(rows_v[b], acc.at[dst_v[q]], sem_sc[b],
                             add=True)

        def wait_scat(b):
            pltpu.make_async_copy(rows_v[b], acc.at[pl.ds(0, K)],
                                  sem_sc[b]).wait()

        def inner(i, b, q, has_prev, nxt, nxt2):
            if has_prev:
                wait_scat(1 - b)
            if nxt2:
                issue_idx(i + 2, (q + 2) % 4)
            if nxt:
                wait_idx((q + 1) % 4)
                issue_fetch(i + 1, 1 - b, (q + 1) % 4)
            wait_fetch(b)

            def mulrow(j, c2):
                for c in range(8):
                    rows_v[b][j, pl.ds(c * L, L)] = (
                        rows_v[b][j, pl.ds(c * L, L)]
                        * gate_v[b][j, pl.ds(c * L, L)])
                return c2
            lax.fori_loop(0, K, mulrow, 0)
            if with_count:
                for g in range(K // L):
                    d = dst_v[q][pl.ds(g * L, L)]
                    plsc.addupdate_scatter(
                        hist_v, [lax.shift_right_logical(d, 7),
                                 lax.bitwise_and(d, 127)], onef)
            issue_scat(b, q)

        n = n_chunks
        issue_idx(0, 0)
        issue_idx(1, 1)
        wait_idx(0)
        issue_fetch(0, 0, 0)
        for i in range(4):
            inner(i, i % 2, i % 4, i >= 1, True, True)
        nloop = (n - 8) // 4

        def quad(j, carry):
            i = 4 * j
            for k in range(4):
                inner(i + k, k % 2, k % 4, True, True, True)
            return carry
        lax.fori_loop(1, 1 + nloop, quad, 0)
        for i in range(4 + 4 * nloop, n):
            inner(i, i % 2, i % 4, True, i + 1 < n, i + 2 < n)
        wait_scat((n - 1) % 2)

        if with_count:
            pltpu.sync_copy(hist_v, cacc.at[iota_v], add=True)
        plsc.subcore_barrier()
        pltpu.sync_copy(acc.at[pl.ds(r0, rows_per_tile)],
                        out_hbm.at[pl.ds(cid * NSP + r0, rows_per_tile)])
        if with_count:
            @pl.when(sid < NR // ZR)
            def _():
                pltpu.sync_copy(cacc.at[pl.ds(sid * ZR, ZR)],
                                cnt_hbm.at[pl.ds(cid * NR + sid * ZR, ZR)])

    return kern, NSP


def _tc_mlp(inputs, w0s, b0, w1, b1, w2, b2, last_sigmoid, out_dtype=_f32):
    N = inputs[0].shape[0]
    B = _pick_chunk(N, 2560)
    n_in = len(inputs)
    dout = w2.shape[1]

    bf = jnp.bfloat16

    def body(*refs):
        xs = refs[:n_in]
        w0r = refs[n_in:2 * n_in]
        b0r, w1r, b1r, w2r, b2r, outr = refs[2 * n_in:]
        l1 = b0r[...]
        for xr, wr in zip(xs, w0r):
            l1 = l1 + jnp.dot(xr[...], wr[...],
                              preferred_element_type=_f32)
        l1 = jnp.tanh(l1)
        l2 = jnp.tanh(jnp.dot(l1.astype(bf), w1r[...].astype(bf),
                              preferred_element_type=_f32) + b1r[...])
        o = jnp.dot(l2.astype(bf), w2r[...].astype(bf),
                    preferred_element_type=_f32) + b2r[...]
        if last_sigmoid:
            o = jax.nn.sigmoid(o)
        outr[...] = o.astype(out_dtype)

    full = lambda a: pl.BlockSpec(a.shape, lambda i: (0,) * a.ndim)
    in_specs = ([pl.BlockSpec((B, a.shape[1]), lambda i: (i, 0))
                 for a in inputs]
                + [full(w) for w in w0s]
                + [full(b0), full(w1), full(b1), full(w2), full(b2)])
    return pl.pallas_call(
        body,
        grid=(N // B,),
        in_specs=in_specs,
        out_specs=pl.BlockSpec((B, dout), lambda i: (i, 0)),
        out_shape=jax.ShapeDtypeStruct((N, dout), out_dtype),
    )(*inputs, *w0s, b0, w1, b1, w2, b2)


def _tc_final(pos, h, x, pu, px, pcnt, w0s, b0, w1, b1, w2, b2):
    NSs = pos.shape[0]
    B = _pick_chunk(NSs, 2560)
    w0p, w0h, w0u, w0m, w0x = w0s

    def body(posr, hr, xr, pu0, pu1, px0, px1, c0, c1,
             w0pr, w0hr, w0ur, w0mr, w0xr, b0r, w1r, b1r, w2r, b2r, outr):
        su = pu0[...] + pu1[...]
        sx = px0[...] + px1[...]
        cnt = (c0[...] + c1[...])[:, 0:1]
        mean = sx / jnp.maximum(cnt, 1.0)
        l1 = (jnp.dot(posr[...], w0pr[...], preferred_element_type=_f32)
              + jnp.dot(hr[...], w0hr[...], preferred_element_type=_f32)
              + jnp.dot(su, w0ur[...], preferred_element_type=_f32)
              + jnp.dot(mean, w0mr[...], preferred_element_type=_f32)
              + jnp.dot(xr[...], w0xr[...], preferred_element_type=_f32)
              + b0r[...])
        l1 = jnp.tanh(l1)
        l2 = jnp.tanh(jnp.dot(l1, w1r[...], preferred_element_type=_f32)
                      + b1r[...])
        outr[...] = (jnp.dot(l2, w2r[...], preferred_element_type=_f32)
                     + b2r[...])

    full = lambda a: pl.BlockSpec(a.shape, lambda i: (0,) * a.ndim)
    row = lambda a: pl.BlockSpec((B, a.shape[1]), lambda i: (i, 0))
    pu0, pu1 = pu[0], pu[1]
    px0, px1 = px[0], px[1]
    c0, c1 = pcnt[0], pcnt[1]
    args = [pos, h, x, pu0, pu1, px0, px1, c0, c1]
    wargs = [w0p, w0h, w0u, w0m, w0x, b0, w1, b1, w2, b2]
    return pl.pallas_call(
        body,
        grid=(NSs // B,),
        in_specs=[row(a) for a in args] + [full(w) for w in wargs],
        out_specs=pl.BlockSpec((B, 128), lambda i: (i, 0)),
        out_shape=jax.ShapeDtypeStruct((NSs, 128), _f32),
    )(*args, *wargs)


def kernel(h, x, u, pos_state, pos_action, a2s_src, a2s_dst, a2s_dis,
           s2s_src, s2s_dst, s2s_dis,
           u2h_dis_W0, u2h_dis_b0, u2h_dis_W1, u2h_dis_b1, u2h_dis_W2,
           u2h_dis_b2,
           u2h_u_W0, u2h_u_b0, u2h_u_W1, u2h_u_b1, u2h_u_W2, u2h_u_b2,
           x2h_dis_W0, x2h_dis_b0, x2h_dis_W1, x2h_dis_b1, x2h_dis_W2,
           x2h_dis_b2,
           x2h_x_W0, x2h_x_b0, x2h_x_W1, x2h_x_b1, x2h_x_W2, x2h_x_b2,
           hupd_W0, hupd_b0, hupd_W1, hupd_b1, hupd_W2, hupd_b2):
    NSs, HID = h.shape
    NAa = u.shape[0]
    E = a2s_src.shape[0]

    r2 = lambda b: b.reshape(1, -1)

    posg = _make_posgather(E, NSs, NAa)
    l1_a, l1_s = posg(pos_state.reshape(-1), pos_action.reshape(-1),
                      a2s_src, a2s_dst, a2s_dis.reshape(-1),
                      s2s_src, s2s_dst, s2s_dis.reshape(-1))
    l1_a = l1_a.reshape(E, 8)
    l1_s = l1_s.reshape(E, 8)

    node_u = _tc_mlp([u], [u2h_u_W0], r2(u2h_u_b0), u2h_u_W1, r2(u2h_u_b1),
                     u2h_u_W2, r2(u2h_u_b2), False)
    node_x = _tc_mlp([x, h], [x2h_x_W0[:x.shape[1]], x2h_x_W0[x.shape[1]:]],
                     r2(x2h_x_b0), x2h_x_W1, r2(x2h_x_b1),
                     x2h_x_W2, r2(x2h_x_b2), False)
    w0pad_u = jnp.zeros((8, u2h_dis_W0.shape[1]), _f32).at[:5].set(u2h_dis_W0)
    w0pad_x = jnp.zeros((8, x2h_dis_W0.shape[1]), _f32).at[:5].set(x2h_dis_W0)

    K = 16
    per_w0 = E // NW
    per_w = -(-per_w0 // K) * K
    pad = per_w - per_w0

    def _pad_e(a, fill):
        if pad == 0:
            return a
        return jnp.pad(a.reshape((NW, per_w0) + a.shape[1:]),
                       ((0, 0), (0, pad)) + ((0, 0),) * (a.ndim - 1),
                       constant_values=fill).reshape((-1,) + a.shape[1:])

    E2 = per_w * NW
    l1_a = _pad_e(l1_a, 0.0)
    l1_s = _pad_e(l1_s, 0.0)
    gate_u = _tc_mlp([l1_a], [w0pad_u], r2(u2h_dis_b0), u2h_dis_W1,
                     r2(u2h_dis_b1), u2h_dis_W2, r2(u2h_dis_b2), True)
    gate_x = _tc_mlp([l1_s], [w0pad_x], r2(x2h_dis_b0), x2h_dis_W1,
                     r2(x2h_dis_b1), x2h_dis_W2, r2(x2h_dis_b2), True)

    scat_u, NSP = _make_scatter(E2, NSs, False)
    pu, = scat_u(node_u, gate_u, _pad_e(a2s_src, 0), _pad_e(a2s_dst, NSs))
    scat_x, _ = _make_scatter(E2, NSs, True)
    px, pcnt = scat_x(node_x, gate_x, _pad_e(s2s_src, 0),
                      _pad_e(s2s_dst, NSs))

    pu = pu.reshape(2, NSP, 128)[:, :NSs]
    px = px.reshape(2, NSP, 128)[:, :NSs]
    pcnt = pcnt.reshape(2, -1)[:, :NSs, None]
    w0s = []
    off = 0
    for d in (2, HID, HID, HID, x.shape[1]):
        w0s.append(hupd_W0[off:off + d])
        off += d
    return _tc_final(pos_state, h, x, pu, px, pcnt, w0s, r2(hupd_b0),
                     hupd_W1, r2(hupd_b1), hupd_W2, r2(hupd_b2))


# speedup vs baseline: 1.1017x; 1.1017x over previous
import functools

import jax
import jax.numpy as jnp
import numpy as np
from jax import lax
from jax.experimental import pallas as pl
from jax.experimental.pallas import tpu as pltpu
from jax.experimental.pallas import tpu_sc as plsc

NCORES = 2
NSUB = 16
NW = NCORES * NSUB
L = 16

_f32 = jnp.float32
_i32 = jnp.int32


def _pick_chunk(n, hi, lo=8):
    for k in range(hi, lo - 1, -8):
        if n % k == 0:
            return k
    raise ValueError(f"no chunk for {n}")


def _pick_div(n, hi):
    for k in range(hi, 0, -1):
        if n % k == 0:
            return k
    return 1


def _sc_mesh():
    return plsc.VectorSubcoreMesh(
        core_axis_name="c", subcore_axis_name="s",
        num_cores=NCORES, num_subcores=NSUB)


def _make_posgather(E, NSs, NAa):
    per_w = E // NW
    K = _pick_chunk(per_w, 2000)
    n_chunks = per_w // K
    n_grp = K // L

    @functools.partial(
        pl.kernel,
        out_type=[jax.ShapeDtypeStruct((E * 8,), _f32),
                  jax.ShapeDtypeStruct((E * 8,), _f32)],
        mesh=_sc_mesh(),
        compiler_params=pltpu.CompilerParams(needs_layout_passes=False),
        scratch_types=[
            pltpu.VMEM((2 * NSs,), _f32),
            pltpu.VMEM((2 * NAa,), _f32),
            [pltpu.VMEM((K,), _i32)] * 2,
            [pltpu.VMEM((K,), _i32)] * 2,
            [pltpu.VMEM((K,), _f32)] * 2,
            [pltpu.VMEM((K * 8,), _f32)] * 2,
            [pltpu.SemaphoreType.DMA] * 2,
            [pltpu.SemaphoreType.DMA] * 2,
        ],
    )
    def kern(ps_hbm, pa_hbm, a_src_hbm, a_dst_hbm, a_dis_hbm,
             s_src_hbm, s_dst_hbm, s_dis_hbm,
             out_a_hbm, out_s_hbm,
             ps_v, pa_v, srci_v, dsti_v, dis_v, stage_v, sem_in, sem_out):
        cid = lax.axis_index("c")
        sid = lax.axis_index("s")
        wid = sid * NCORES + cid
        pltpu.sync_copy(ps_hbm, ps_v)
        pltpu.sync_copy(pa_hbm, pa_v)
        lanes = lax.iota(_i32, L)
        zeros_f = jnp.zeros((L,), _f32)
        ones_i = jnp.ones((L,), _i32)

        def do_edges(src_hbm, dst_hbm, dis_hbm, out_hbm, src_tab):
            def issue_in(i, b):
                base = wid * per_w + i * K
                return (
                    pltpu.async_copy(src_hbm.at[pl.ds(base, K)],
                                     srci_v[b], sem_in[b]),
                    pltpu.async_copy(dst_hbm.at[pl.ds(base, K)],
                                     dsti_v[b], sem_in[b]),
                    pltpu.async_copy(dis_hbm.at[pl.ds(base, K)],
                                     dis_v[b], sem_in[b]),
                )

            pending_in = {0: issue_in(0, 0)}
            pending_out = {}
            for i in range(n_chunks):
                b = i % 2
                if i + 1 < n_chunks:
                    pending_in[i + 1] = issue_in(i + 1, 1 - b)
                for d in pending_in.pop(i):
                    d.wait()
                if i - 2 in pending_out:
                    pending_out.pop(i - 2).wait()

                def grp(g, carry):
                    rid8 = (lanes + g * L) * 8
                    srci = srci_v[b][pl.ds(g * L, L)] * 2
                    dsti = dsti_v[b][pl.ds(g * L, L)] * 2
                    sx = plsc.load_gather(src_tab, [srci])
                    sy = plsc.load_gather(src_tab, [srci + ones_i])
                    dx = plsc.load_gather(ps_v, [dsti])
                    dy = plsc.load_gather(ps_v, [dsti + ones_i])
                    dv = dis_v[b][pl.ds(g * L, L)]
                    for col, val in ((0, sx), (1, sy), (2, dx), (3, dy),
                                     (4, dv), (5, zeros_f), (6, zeros_f),
                                     (7, zeros_f)):
                        plsc.store_scatter(
                            stage_v[b], [rid8 + col], val)
                    return carry
                lax.fori_loop(0, n_grp, grp, 0)

                base = (wid * per_w + i * K) * 8
                pending_out[i] = pltpu.async_copy(
                    stage_v[b], out_hbm.at[pl.ds(base, K * 8)], sem_out[b])
            for d in pending_out.values():
                d.wait()

        do_edges(a_src_hbm, a_dst_hbm, a_dis_hbm, out_a_hbm, pa_v)
        do_edges(s_src_hbm, s_dst_hbm, s_dis_hbm, out_s_hbm, ps_v)

    return kern


def _make_scatter(E, NSs, with_count):
    K = 16
    per_w = E // NW
    assert per_w % K == 0
    n_chunks = per_w // K
    assert n_chunks >= 16
    NSP = -(-NSs // 128) * 128
    assert (NSP // NSUB) % 8 == 0
    rows_per_tile = NSP // NSUB
    ZR = 8

    NR = -(-(NSP // 128) // 8) * 8
    out_type = [jax.ShapeDtypeStruct((2 * NSP, 128), _f32)]
    scratch = [
        [pltpu.VMEM((K,), _i32)] * 4,
        [pltpu.VMEM((K,), _i32)] * 4,
        [pltpu.VMEM((K, 128), _f32)] * 4,
        [pltpu.VMEM((K, 128), _f32)] * 4,
        [pltpu.VMEM((K,), _i32)] * 2,
        pltpu.VMEM_SHARED((NSP, 128), _f32),
        [pltpu.SemaphoreType.DMA] * 4,
        [pltpu.SemaphoreType.DMA] * 4,
        [pltpu.SemaphoreType.DMA] * 4,
        [pltpu.SemaphoreType.DMA] * 2,
    ]
    if with_count:
        out_type.append(jax.ShapeDtypeStruct((2 * NR, 128), _f32))
        scratch += [
            pltpu.VMEM((NR, 128), _f32),
            pltpu.VMEM((NR,), _i32),
            pltpu.VMEM_SHARED((NR, 128), _f32),
        ]

    @functools.partial(
        pl.kernel, out_type=out_type, mesh=_sc_mesh(),
        compiler_params=pltpu.CompilerParams(needs_layout_passes=False),
        scratch_types=scratch)
    def kern(feat_hbm, gate_hbm, src_hbm, dst_hbm, *rest):
        if with_count:
            (out_hbm, cnt_hbm, src_v, dst_v, rows_v, gate_v, dst_sc, acc,
             sem_idx, sem_rows, sem_gate, sem_sc,
             hist_v, iota_v, cacc) = rest
        else:
            (out_hbm, src_v, dst_v, rows_v, gate_v, dst_sc, acc,
             sem_idx, sem_rows, sem_gate, sem_sc) = rest
        cid = lax.axis_index("c")
        sid = lax.axis_index("s")
        wid = sid * NCORES + cid
        r0 = sid * rows_per_tile
        zf = jnp.zeros((L,), _f32)
        onef = jnp.ones((L,), _f32)
        lanes = lax.iota(_i32, L)

        def zero_rows0(j, carry):
            for c in range(8):
                rows_v[0][j, pl.ds(c * L, L)] = zf
            return carry
        lax.fori_loop(0, K, zero_rows0, 0)

        def zero_acc(j, carry):
            pltpu.sync_copy(rows_v[0].at[pl.ds(0, ZR)],
                            acc.at[pl.ds(r0 + j * ZR, ZR)])
            return carry
        lax.fori_loop(0, rows_per_tile // ZR, zero_acc, 0)

        if with_count:
            def zero_hist(j, carry):
                for c in range(8):
                    hist_v[j, pl.ds(c * L, L)] = zf
                return carry
            lax.fori_loop(0, NR, zero_hist, 0)
            for g in range(NR // L):
                iota_v[pl.ds(g * L, L)] = lanes + g * L

            @pl.when(sid < NR // ZR)
            def _():
                pltpu.sync_copy(rows_v[0].at[pl.ds(0, ZR)],
                                cacc.at[pl.ds(sid * ZR, ZR)])

        plsc.subcore_barrier()

        def issue_idx(i, q):
            base = wid * per_w + i * K
            pltpu.async_copy(src_hbm.at[pl.ds(base, K)], src_v[q],
                             sem_idx[q])
            pltpu.async_copy(dst_hbm.at[pl.ds(base, K)], dst_v[q],
                             sem_idx[q])

        def wait_idx(q):
            pltpu.make_async_copy(src_hbm.at[pl.ds(0, K)], src_v[q],
                                  sem_idx[q]).wait()
            pltpu.make_async_copy(dst_hbm.at[pl.ds(0, K)], dst_v[q],
                                  sem_idx[q]).wait()

        def issue_fetch(i, q):
            base = wid * per_w + i * K
            pltpu.async_copy(feat_hbm.at[src_v[q]], rows_v[q], sem_rows[q])
            pltpu.async_copy(gate_hbm.at[pl.ds(base, K)], gate_v[q],
                             sem_gate[q])

        def wait_fetch(q):
            pltpu.make_async_copy(feat_hbm.at[pl.ds(0, K)], rows_v[q],
                                  sem_rows[q]).wait()
            pltpu.make_async_copy(gate_hbm.at[pl.ds(0, K)], gate_v[q],
                                  sem_gate[q]).wait()

        def issue_scat(q, b):
            pltpu.async_copy(rows_v[q], acc.at[dst_sc[b]], sem_sc[b],
                             add=True)

        def wait_scat(b):
            pltpu.make_async_copy(rows_v[0], acc.at[pl.ds(0, K)],
                                  sem_sc[b]).wait()

        def inner(i, q, b, w_scat, nxt2, nxt3):
            if w_scat:
                wait_scat(b)
            if nxt2:
                wait_idx((q + 2) % 4)
                issue_fetch(i + 2, (q + 2) % 4)
            if nxt3:
                issue_idx(i + 3, (q + 3) % 4)
            wait_fetch(q)

            def mulrow(j, c2):
                for c in range(8):
                    rows_v[q][j, pl.ds(c * L, L)] = (
                        rows_v[q][j, pl.ds(c * L, L)]
                        * gate_v[q][j, pl.ds(c * L, L)])
                return c2
            lax.fori_loop(0, K, mulrow, 0)
            if with_count:
                for g in range(K // L):
                    d = dst_v[q][pl.ds(g * L, L)]
                    plsc.addupdate_scatter(
                        hist_v, [lax.shift_right_logical(d, 7),
                                 lax.bitwise_and(d, 127)], onef)
            for g in range(K // L):
                dst_sc[b][pl.ds(g * L, L)] = dst_v[q][pl.ds(g * L, L)]
            issue_scat(q, b)

        n = n_chunks
        issue_idx(0, 0)
        issue_idx(1, 1)
        issue_idx(2, 2)
        wait_idx(0)
        issue_fetch(0, 0)
        wait_idx(1)
        issue_fetch(1, 1)
        for i in range(4):
            inner(i, i % 4, i % 2, i >= 2, True, True)
        nloop = (n - 8) // 4

        def quad(j, carry):
            i = 4 * j
            for k in range(4):
                inner(i + k, k % 4, k % 2, True, True, True)
            return carry
        lax.fori_loop(1, 1 + nloop, quad, 0)
        for i in range(4 + 4 * nloop, n):
            inner(i, i % 4, i % 2, True, i + 2 < n, i + 3 < n)
        wait_scat((n - 2) % 2)
        wait_scat((n - 1) % 2)

        if with_count:
            pltpu.sync_copy(hist_v, cacc.at[iota_v], add=True)
        plsc.subcore_barrier()
        pltpu.sync_copy(acc.at[pl.ds(r0, rows_per_tile)],
                        out_hbm.at[pl.ds(cid * NSP + r0, rows_per_tile)])
        if with_count:
            @pl.when(sid < NR // ZR)
            def _():
                pltpu.sync_copy(cacc.at[pl.ds(sid * ZR, ZR)],
                                cnt_hbm.at[pl.ds(cid * NR + sid * ZR, ZR)])

    return kern, NSP


def _tc_mlp(inputs, w0s, b0, w1, b1, w2, b2, last_sigmoid, out_dtype=_f32):
    N = inputs[0].shape[0]
    B = _pick_chunk(N, 2560)
    n_in = len(inputs)
    dout = w2.shape[1]

    bf = jnp.bfloat16

    def body(*refs):
        xs = refs[:n_in]
        w0r = refs[n_in:2 * n_in]
        b0r, w1r, b1r, w2r, b2r, outr = refs[2 * n_in:]
        l1 = b0r[...]
        for xr, wr in zip(xs, w0r):
            l1 = l1 + jnp.dot(xr[...], wr[...],
                              preferred_element_type=_f32)
        l1 = jnp.tanh(l1)
        l2 = jnp.tanh(jnp.dot(l1.astype(bf), w1r[...].astype(bf),
                              preferred_element_type=_f32) + b1r[...])
        o = jnp.dot(l2.astype(bf), w2r[...].astype(bf),
                    preferred_element_type=_f32) + b2r[...]
        if last_sigmoid:
            o = jax.nn.sigmoid(o)
        outr[...] = o.astype(out_dtype)

    full = lambda a: pl.BlockSpec(a.shape, lambda i: (0,) * a.ndim)
    in_specs = ([pl.BlockSpec((B, a.shape[1]), lambda i: (i, 0))
                 for a in inputs]
                + [full(w) for w in w0s]
                + [full(b0), full(w1), full(b1), full(w2), full(b2)])
    return pl.pallas_call(
        body,
        grid=(N // B,),
        in_specs=in_specs,
        out_specs=pl.BlockSpec((B, dout), lambda i: (i, 0)),
        out_shape=jax.ShapeDtypeStruct((N, dout), out_dtype),
    )(*inputs, *w0s, b0, w1, b1, w2, b2)


def _tc_final(pos, h, x, pu, px, pcnt, w0s, b0, w1, b1, w2, b2):
    NSs = pos.shape[0]
    B = _pick_chunk(NSs, 2560)
    w0p, w0h, w0u, w0m, w0x = w0s

    def body(posr, hr, xr, pu0, pu1, px0, px1, c0, c1,
             w0pr, w0hr, w0ur, w0mr, w0xr, b0r, w1r, b1r, w2r, b2r, outr):
        su = pu0[...] + pu1[...]
        sx = px0[...] + px1[...]
        cnt = (c0[...] + c1[...])[:, 0:1]
        mean = sx / jnp.maximum(cnt, 1.0)
        l1 = (jnp.dot(posr[...], w0pr[...], preferred_element_type=_f32)
              + jnp.dot(hr[...], w0hr[...], preferred_element_type=_f32)
              + jnp.dot(su, w0ur[...], preferred_element_type=_f32)
              + jnp.dot(mean, w0mr[...], preferred_element_type=_f32)
              + jnp.dot(xr[...], w0xr[...], preferred_element_type=_f32)
              + b0r[...])
        l1 = jnp.tanh(l1)
        l2 = jnp.tanh(jnp.dot(l1, w1r[...], preferred_element_type=_f32)
                      + b1r[...])
        outr[...] = (jnp.dot(l2, w2r[...], preferred_element_type=_f32)
                     + b2r[...])

    full = lambda a: pl.BlockSpec(a.shape, lambda i: (0,) * a.ndim)
    row = lambda a: pl.BlockSpec((B, a.shape[1]), lambda i: (i, 0))
    pu0, pu1 = pu[0], pu[1]
    px0, px1 = px[0], px[1]
    c0, c1 = pcnt[0], pcnt[1]
    args = [pos, h, x, pu0, pu1, px0, px1, c0, c1]
    wargs = [w0p, w0h, w0u, w0m, w0x, b0, w1, b1, w2, b2]
    return pl.pallas_call(
        body,
        grid=(NSs // B,),
        in_specs=[row(a) for a in args] + [full(w) for w in wargs],
        out_specs=pl.BlockSpec((B, 128), lambda i: (i, 0)),
        out_shape=jax.ShapeDtypeStruct((NSs, 128), _f32),
    )(*args, *wargs)


def kernel(h, x, u, pos_state, pos_action, a2s_src, a2s_dst, a2s_dis,
           s2s_src, s2s_dst, s2s_dis,
           u2h_dis_W0, u2h_dis_b0, u2h_dis_W1, u2h_dis_b1, u2h_dis_W2,
           u2h_dis_b2,
           u2h_u_W0, u2h_u_b0, u2h_u_W1, u2h_u_b1, u2h_u_W2, u2h_u_b2,
           x2h_dis_W0, x2h_dis_b0, x2h_dis_W1, x2h_dis_b1, x2h_dis_W2,
           x2h_dis_b2,
           x2h_x_W0, x2h_x_b0, x2h_x_W1, x2h_x_b1, x2h_x_W2, x2h_x_b2,
           hupd_W0, hupd_b0, hupd_W1, hupd_b1, hupd_W2, hupd_b2):
    NSs, HID = h.shape
    NAa = u.shape[0]
    E = a2s_src.shape[0]

    r2 = lambda b: b.reshape(1, -1)

    posg = _make_posgather(E, NSs, NAa)
    l1_a, l1_s = posg(pos_state.reshape(-1), pos_action.reshape(-1),
                      a2s_src, a2s_dst, a2s_dis.reshape(-1),
                      s2s_src, s2s_dst, s2s_dis.reshape(-1))
    l1_a = l1_a.reshape(E, 8)
    l1_s = l1_s.reshape(E, 8)

    node_u = _tc_mlp([u], [u2h_u_W0], r2(u2h_u_b0), u2h_u_W1, r2(u2h_u_b1),
                     u2h_u_W2, r2(u2h_u_b2), False)
    node_x = _tc_mlp([x, h], [x2h_x_W0[:x.shape[1]], x2h_x_W0[x.shape[1]:]],
                     r2(x2h_x_b0), x2h_x_W1, r2(x2h_x_b1),
                     x2h_x_W2, r2(x2h_x_b2), False)
    w0pad_u = jnp.zeros((8, u2h_dis_W0.shape[1]), _f32).at[:5].set(u2h_dis_W0)
    w0pad_x = jnp.zeros((8, x2h_dis_W0.shape[1]), _f32).at[:5].set(x2h_dis_W0)

    K = 16
    per_w0 = E // NW
    per_w = -(-per_w0 // K) * K
    pad = per_w - per_w0

    def _pad_e(a, fill):
        if pad == 0:
            return a
        return jnp.pad(a.reshape((NW, per_w0) + a.shape[1:]),
                       ((0, 0), (0, pad)) + ((0, 0),) * (a.ndim - 1),
                       constant_values=fill).reshape((-1,) + a.shape[1:])

    E2 = per_w * NW
    l1_a = _pad_e(l1_a, 0.0)
    l1_s = _pad_e(l1_s, 0.0)
    gate_u = _tc_mlp([l1_a], [w0pad_u], r2(u2h_dis_b0), u2h_dis_W1,
                     r2(u2h_dis_b1), u2h_dis_W2, r2(u2h_dis_b2), True)
    gate_x = _tc_mlp([l1_s], [w0pad_x], r2(x2h_dis_b0), x2h_dis_W1,
                     r2(x2h_dis_b1), x2h_dis_W2, r2(x2h_dis_b2), True)

    scat_u, NSP = _make_scatter(E2, NSs, False)
    pu, = scat_u(node_u, gate_u, _pad_e(a2s_src, 0), _pad_e(a2s_dst, NSs))
    scat_x, _ = _make_scatter(E2, NSs, True)
    px, pcnt = scat_x(node_x, gate_x, _pad_e(s2s_src, 0),
                      _pad_e(s2s_dst, NSs))

    pu = pu.reshape(2, NSP, 128)[:, :NSs]
    px = px.reshape(2, NSP, 128)[:, :NSs]
    pcnt = pcnt.reshape(2, -1)[:, :NSs, None]
    w0s = []
    off = 0
    for d in (2, HID, HID, HID, x.shape[1]):
        w0s.append(hupd_W0[off:off + d])
        off += d
    return _tc_final(pos_state, h, x, pu, px, pcnt, w0s, r2(hupd_b0),
                     hupd_W1, r2(hupd_b1), hupd_W2, r2(hupd_b2))
